# Initial kernel scaffold; baseline (speedup 1.0000x reference)
import jax
import jax.numpy as jnp
from jax.experimental import pallas as pl


def kernel(label, cls_ctx, token_prefix, token_suffix):
    raise NotImplementedError("write your pallas kernel here")


# speedup vs baseline: 1.0120x; 1.0120x over previous
import functools

import jax
import jax.numpy as jnp
from jax import lax
from jax.experimental import pallas as pl
from jax.experimental.pallas import tpu as pltpu
from jax.experimental.pallas import tpu_sc as plsc

B = 4096
V = 100000
NCC = 4
D = 512
SEQ = 77
P = 5
S = SEQ - P - NCC

NC, NS = 2, 16
NW = NC * NS
BPW = B // NW
CHUNK = 16
NCHUNK = BPW // CHUNK


def _sc_gather(label, cls_ctx):
    mesh = plsc.VectorSubcoreMesh(core_axis_name="c", subcore_axis_name="s")

    @functools.partial(
        pl.kernel,
        out_type=jax.ShapeDtypeStruct((B, NCC, D), jnp.float32),
        mesh=mesh,
        scratch_types=[
            pltpu.VMEM((BPW,), jnp.int32),
            pltpu.VMEM((CHUNK, NCC, D), jnp.float32),
            pltpu.VMEM((CHUNK, NCC, D), jnp.float32),
            pltpu.SemaphoreType.DMA,
            pltpu.SemaphoreType.DMA,
            pltpu.SemaphoreType.DMA,
            pltpu.SemaphoreType.DMA,
        ],
    )
    def run(label_hbm, cls_hbm, out_hbm, idx_v, buf0, buf1, g0, g1, s0, s1):
        wid = lax.axis_index("s") * NC + lax.axis_index("c")
        base = wid * BPW
        pltpu.sync_copy(label_hbm.at[pl.ds(base, BPW)], idx_v)
        bufs = (buf0, buf1)
        gsems = (g0, g1)
        ssems = (s0, s1)

        def store_cp(c):
            p = c & 1
            return pltpu.make_async_copy(
                bufs[p], out_hbm.at[pl.ds(base + c * CHUNK, CHUNK)], ssems[p])

        for c in range(NCHUNK):
            p = c & 1
            if c >= 2:
                store_cp(c - 2).wait()
            gcp = pltpu.make_async_copy(
                cls_hbm.at[idx_v.at[pl.ds(c * CHUNK, CHUNK)]],
                bufs[p],
                gsems[p],
            )
            gcp.start()
            gcp.wait()
            store_cp(c).start()
        store_cp(NCHUNK - 2).wait()
        store_cp(NCHUNK - 1).wait()

    return run(label, cls_ctx)


BB = 128


def _tc_assemble(gathered, prefix, suffix):
    def body(g_ref, pre_ref, suf_ref, out_ref):
        out_ref[:, 0:P, :] = jnp.broadcast_to(pre_ref[...], (BB, P, D))
        out_ref[:, P:P + NCC, :] = g_ref[...]
        out_ref[:, P + NCC:SEQ, :] = jnp.broadcast_to(suf_ref[...], (BB, S, D))

    return pl.pallas_call(
        body,
        grid=(B // BB,),
        in_specs=[
            pl.BlockSpec((BB, NCC, D), lambda i: (i, 0, 0)),
            pl.BlockSpec((1, P, D), lambda i: (0, 0, 0)),
            pl.BlockSpec((1, S, D), lambda i: (0, 0, 0)),
        ],
        out_specs=pl.BlockSpec((BB, SEQ, D), lambda i: (i, 0, 0)),
        out_shape=jax.ShapeDtypeStruct((B, SEQ, D), jnp.float32),
    )(gathered, prefix, suffix)


def kernel(label, cls_ctx, token_prefix, token_suffix):
    gathered = _sc_gather(label.astype(jnp.int32), cls_ctx)
    return _tc_assemble(gathered, token_prefix, token_suffix)
